# baseline (speedup 1.0000x reference)
import jax
import jax.numpy as jnp
from jax import lax
from jax.experimental import pallas as pl
from jax.experimental.pallas import tpu as pltpu

B = 16
N = 10000
H = 128
F = 4
IN = 2 * H + 2 * F
G3 = 3 * H


def _body(uid_ref, iid_ref, uf_ref, itf_ref,
          wu_ref, bihu_ref, bhhu_ref,
          wi_ref, bihi_ref, bhhi_ref,
          ublk_ref, iblk_ref, umem_ref, imem_ref,
          nu_ref, ni_ref, uout_ref, iout_ref,
          ue_ref, ie_ref, xu_ref, xi_ref, sem_g):
    b = pl.program_id(0)

    @pl.when(b == 0)
    def _compute():
        gath = [pltpu.make_async_copy(umem_ref.at[k, uid_ref[k]], ue_ref.at[k],
                                      sem_g) for k in range(B)]
        gath += [pltpu.make_async_copy(imem_ref.at[k, iid_ref[k]], ie_ref.at[k],
                                       sem_g) for k in range(B)]
        for c in gath:
            c.start()
        for c in gath:
            c.wait()

        ue = ue_ref[...]
        ie = ie_ref[...]
        uf = uf_ref[...]
        itf = itf_ref[...]

        xu_ref[:, 0:H] = ue
        xu_ref[:, H:H + F] = uf
        xu_ref[:, H + F:H + F + H] = ie
        xu_ref[:, H + F + H:IN] = itf
        xi_ref[:, 0:H] = ie
        xi_ref[:, H:H + F] = itf
        xi_ref[:, H + F:H + F + H] = ue
        xi_ref[:, H + F + H:IN] = uf

        def gru(x_ref, w_ref, bih_ref, bhh_ref):
            gx = lax.dot_general(x_ref[...], w_ref[...],
                                 (((1,), (1,)), ((), ())),
                                 preferred_element_type=jnp.float32)
            gx = gx + bih_ref[...]
            bhh = bhh_ref[...]
            g = gx + bhh
            r = jax.nn.sigmoid(g[:, :H])
            z = jax.nn.sigmoid(g[:, H:2 * H])
            n = jnp.tanh(gx[:, 2 * H:] + r * bhh[:, 2 * H:])
            out = (1.0 - z) * n
            nrm = jnp.sqrt(jnp.sum(out * out, axis=1, keepdims=True))
            return out / jnp.maximum(nrm, 1e-12)

        nu_ref[...] = gru(xu_ref, wu_ref, bihu_ref, bhhu_ref)
        ni_ref[...] = gru(xi_ref, wi_ref, bihi_ref, bhhi_ref)

    uout_ref[...] = ublk_ref[...]
    iout_ref[...] = iblk_ref[...]

    uid = uid_ref[b]
    iid = iid_ref[b]
    uout_ref[0, pl.ds(uid, 1), :] = nu_ref[pl.ds(b, 1), :]
    iout_ref[0, pl.ds(iid, 1), :] = ni_ref[pl.ds(b, 1), :]


def kernel(user_ids, item_ids, user_features, item_features, user_memory,
           item_memory, W_ih_u, W_hh_u, b_ih_u, b_hh_u, W_ih_i, W_hh_i,
           b_ih_i, b_hh_i):
    del W_hh_u, W_hh_i
    smem = pl.BlockSpec(memory_space=pltpu.SMEM)
    anym = pl.BlockSpec(memory_space=pltpu.MemorySpace.HBM)
    blk = pl.BlockSpec((1, N, H), lambda b: (b, 0, 0))

    def full(*shape):
        return pl.BlockSpec(shape, lambda b: (0,) * len(shape))

    f32 = jnp.float32
    return pl.pallas_call(
        _body,
        grid=(B,),
        out_shape=(
            jax.ShapeDtypeStruct((B, H), f32),
            jax.ShapeDtypeStruct((B, H), f32),
            jax.ShapeDtypeStruct((B, N, H), f32),
            jax.ShapeDtypeStruct((B, N, H), f32),
        ),
        in_specs=[smem, smem, full(B, F), full(B, F),
                  full(G3, IN), full(1, G3), full(1, G3),
                  full(G3, IN), full(1, G3), full(1, G3),
                  blk, blk, anym, anym],
        out_specs=(
            pl.BlockSpec((B, H), lambda b: (0, 0)),
            pl.BlockSpec((B, H), lambda b: (0, 0)),
            blk,
            blk,
        ),
        scratch_shapes=[
            pltpu.VMEM((B, H), f32),
            pltpu.VMEM((B, H), f32),
            pltpu.VMEM((B, IN), f32),
            pltpu.VMEM((B, IN), f32),
            pltpu.SemaphoreType.DMA,
        ],
    )(user_ids, item_ids, user_features, item_features,
      W_ih_u, b_ih_u.reshape(1, G3), b_hh_u.reshape(1, G3),
      W_ih_i, b_ih_i.reshape(1, G3), b_hh_i.reshape(1, G3),
      user_memory, item_memory, user_memory, item_memory)


# speedup vs baseline: 1.0027x; 1.0027x over previous
import jax
import jax.numpy as jnp
from jax import lax
from jax.experimental import pallas as pl
from jax.experimental.pallas import tpu as pltpu

B = 16
N = 10000
H = 128
F = 4
IN = 2 * H + 2 * F
G3 = 3 * H


def _body(uid_ref, iid_ref,
          gfu_ref, gfi_ref,
          we_ref,
          bhhu_ref, bhhi_ref,
          ublk_ref, iblk_ref, umem_ref, imem_ref,
          nu_ref, ni_ref, uout_ref, iout_ref,
          ue_ref, ie_ref, sem_g):
    b = pl.program_id(0)

    @pl.when(b == 0)
    def _compute():
        gath = [pltpu.make_async_copy(umem_ref.at[k, uid_ref[k]], ue_ref.at[k],
                                      sem_g) for k in range(B)]
        gath += [pltpu.make_async_copy(imem_ref.at[k, iid_ref[k]], ie_ref.at[k],
                                       sem_g) for k in range(B)]
        for c in gath:
            c.start()
        for c in gath:
            c.wait()

        ue = ue_ref[...]
        ie = ie_ref[...]

        def matmul(x, w):
            return lax.dot_general(x, w, (((1,), (1,)), ((), ())),
                                   preferred_element_type=jnp.float32)

        def gru(e1, e2, w1, w2, gf, bhh_ref):
            gx = matmul(e1, w1) + matmul(e2, w2) + gf
            bhh = bhh_ref[...]
            g = gx + bhh
            r = jax.nn.sigmoid(g[:, :H])
            z = jax.nn.sigmoid(g[:, H:2 * H])
            n = jnp.tanh(gx[:, 2 * H:] + r * bhh[:, 2 * H:])
            out = (1.0 - z) * n
            nrm = jnp.sqrt(jnp.sum(out * out, axis=1, keepdims=True))
            return out / jnp.maximum(nrm, 1e-12)

        nu_ref[...] = gru(ue, ie, we_ref[0], we_ref[1], gfu_ref[...], bhhu_ref)
        ni_ref[...] = gru(ie, ue, we_ref[2], we_ref[3], gfi_ref[...], bhhi_ref)

    uout_ref[...] = ublk_ref[...]
    iout_ref[...] = iblk_ref[...]

    uout_ref[0, pl.ds(uid_ref[b], 1), :] = nu_ref[pl.ds(b, 1), :]
    iout_ref[0, pl.ds(iid_ref[b], 1), :] = ni_ref[pl.ds(b, 1), :]


def kernel(user_ids, item_ids, user_features, item_features, user_memory,
           item_memory, W_ih_u, W_hh_u, b_ih_u, b_hh_u, W_ih_i, W_hh_i,
           b_ih_i, b_hh_i):
    del W_hh_u, W_hh_i
    we = jnp.stack([W_ih_u[:, :H], W_ih_u[:, H + F:H + F + H],
                    W_ih_i[:, :H], W_ih_i[:, H + F:H + F + H]])
    gfu = (user_features @ W_ih_u[:, H:H + F].T
           + item_features @ W_ih_u[:, H + F + H:].T + b_ih_u)
    gfi = (item_features @ W_ih_i[:, H:H + F].T
           + user_features @ W_ih_i[:, H + F + H:].T + b_ih_i)
    smem = pl.BlockSpec(memory_space=pltpu.SMEM)
    anym = pl.BlockSpec(memory_space=pltpu.MemorySpace.HBM)
    blk = pl.BlockSpec((1, N, H), lambda b: (b, 0, 0))

    def full(*shape):
        return pl.BlockSpec(shape, lambda b: (0,) * len(shape))

    f32 = jnp.float32
    return pl.pallas_call(
        _body,
        grid=(B,),
        out_shape=(
            jax.ShapeDtypeStruct((B, H), f32),
            jax.ShapeDtypeStruct((B, H), f32),
            jax.ShapeDtypeStruct((B, N, H), f32),
            jax.ShapeDtypeStruct((B, N, H), f32),
        ),
        in_specs=[smem, smem, full(B, G3), full(B, G3), full(4, G3, H),
                  full(1, G3), full(1, G3), blk, blk, anym, anym],
        out_specs=(
            pl.BlockSpec((B, H), lambda b: (0, 0)),
            pl.BlockSpec((B, H), lambda b: (0, 0)),
            blk,
            blk,
        ),
        scratch_shapes=[
            pltpu.VMEM((B, H), f32),
            pltpu.VMEM((B, H), f32),
            pltpu.SemaphoreType.DMA,
        ],
    )(user_ids, item_ids, gfu, gfi, we,
      b_hh_u.reshape(1, G3), b_hh_i.reshape(1, G3),
      user_memory, item_memory, user_memory, item_memory)


# speedup vs baseline: 1.0159x; 1.0131x over previous
import jax
import jax.numpy as jnp
from jax import lax
from jax.experimental import pallas as pl
from jax.experimental.pallas import tpu as pltpu

B = 16
N = 10000
H = 128
F = 4
G3 = 3 * H


def _body(uid_ref, iid_ref, uf_ref, itf_ref, p_ref,
          ublk_ref, iblk_ref, umem_ref, imem_ref,
          nu_ref, ni_ref, uout_ref, iout_ref,
          ue_ref, ie_ref, xu_ref, xi_ref, sem_g):
    b = pl.program_id(0)

    @pl.when(b == 0)
    def _compute():
        gath = [pltpu.make_async_copy(umem_ref.at[k, uid_ref[k]], ue_ref.at[k],
                                      sem_g) for k in range(B)]
        gath += [pltpu.make_async_copy(imem_ref.at[k, iid_ref[k]], ie_ref.at[k],
                                       sem_g) for k in range(B)]
        for c in gath:
            c.start()
        for c in gath:
            c.wait()

        lane = lax.broadcasted_iota(jnp.int32, (1, H), 1)
        ones_cols = jnp.where((lane == 8) | (lane == 9), 1.0, 0.0)
        for k in range(B):
            ru = ones_cols
            ri = ones_cols
            for c in range(F):
                ru = ru + jnp.where(lane == c, uf_ref[k, c], 0.0)
                ru = ru + jnp.where(lane == F + c, itf_ref[k, c], 0.0)
                ri = ri + jnp.where(lane == c, itf_ref[k, c], 0.0)
                ri = ri + jnp.where(lane == F + c, uf_ref[k, c], 0.0)
            xu_ref[k:k + 1, :] = ru
            xi_ref[k:k + 1, :] = ri
        sel_bhh = jnp.broadcast_to(jnp.where(lane == 9, 1.0, 0.0), (B, H))

        ue = ue_ref[...]
        ie = ie_ref[...]

        def matmul(x, w):
            return lax.dot_general(x, w, (((1,), (1,)), ((), ())),
                                   preferred_element_type=jnp.float32)

        def gru(e1, e2, w1, w2, xf, wf):
            g = matmul(e1, w1) + matmul(e2, w2) + matmul(xf, wf)
            bhh = matmul(sel_bhh, wf)
            r = jax.nn.sigmoid(g[:, :H])
            z = jax.nn.sigmoid(g[:, H:2 * H])
            n = jnp.tanh(g[:, 2 * H:] + (r - 1.0) * bhh[:, 2 * H:])
            out = (1.0 - z) * n
            nrm = jnp.sqrt(jnp.sum(out * out, axis=1, keepdims=True))
            return out / jnp.maximum(nrm, 1e-12)

        nu_ref[...] = gru(ue, ie, p_ref[0:G3, :], p_ref[G3:2 * G3, :],
                          xu_ref[...], p_ref[4 * G3:5 * G3, :])
        ni_ref[...] = gru(ie, ue, p_ref[2 * G3:3 * G3, :], p_ref[3 * G3:4 * G3, :],
                          xi_ref[...], p_ref[5 * G3:6 * G3, :])

    uout_ref[...] = ublk_ref[...]
    iout_ref[...] = iblk_ref[...]

    uout_ref[0, pl.ds(uid_ref[b], 1), :] = nu_ref[pl.ds(b, 1), :]
    iout_ref[0, pl.ds(iid_ref[b], 1), :] = ni_ref[pl.ds(b, 1), :]


def kernel(user_ids, item_ids, user_features, item_features, user_memory,
           item_memory, W_ih_u, W_hh_u, b_ih_u, b_hh_u, W_ih_i, W_hh_i,
           b_ih_i, b_hh_i):
    del W_hh_u, W_hh_i
    z = jnp.zeros((G3, H - 10), jnp.float32)
    wf_u = jnp.concatenate([W_ih_u[:, H:H + F], W_ih_u[:, H + F + H:],
                            b_ih_u[:, None], b_hh_u[:, None], z], axis=1)
    wf_i = jnp.concatenate([W_ih_i[:, H:H + F], W_ih_i[:, H + F + H:],
                            b_ih_i[:, None], b_hh_i[:, None], z], axis=1)
    packed = jnp.concatenate([W_ih_u[:, :H], W_ih_u[:, H + F:H + F + H],
                              W_ih_i[:, :H], W_ih_i[:, H + F:H + F + H],
                              wf_u, wf_i], axis=0)
    smem = pl.BlockSpec(memory_space=pltpu.SMEM)
    anym = pl.BlockSpec(memory_space=pltpu.MemorySpace.HBM)
    blk = pl.BlockSpec((1, N, H), lambda b: (b, 0, 0))
    f32 = jnp.float32
    return pl.pallas_call(
        _body,
        grid=(B,),
        out_shape=(
            jax.ShapeDtypeStruct((B, H), f32),
            jax.ShapeDtypeStruct((B, H), f32),
            jax.ShapeDtypeStruct((B, N, H), f32),
            jax.ShapeDtypeStruct((B, N, H), f32),
        ),
        in_specs=[smem, smem, smem, smem,
                  pl.BlockSpec((6 * G3, H), lambda b: (0, 0)),
                  blk, blk, anym, anym],
        out_specs=(
            pl.BlockSpec((B, H), lambda b: (0, 0)),
            pl.BlockSpec((B, H), lambda b: (0, 0)),
            blk,
            blk,
        ),
        scratch_shapes=[
            pltpu.VMEM((B, H), f32),
            pltpu.VMEM((B, H), f32),
            pltpu.VMEM((B, H), f32),
            pltpu.VMEM((B, H), f32),
            pltpu.SemaphoreType.DMA,
        ],
    )(user_ids, item_ids, user_features, item_features, packed,
      user_memory, item_memory, user_memory, item_memory)


# speedup vs baseline: 1.0482x; 1.0318x over previous
import jax
import jax.numpy as jnp
from jax import lax
from jax.experimental import pallas as pl
from jax.experimental.pallas import tpu as pltpu

B = 16
N = 10000
H = 128
F = 4
IN = 2 * H + 2 * F
G3 = 3 * H
C0 = 2 * H


def _body(uid_ref, iid_ref,
          uf_ref, itf_ref,
          wu_ref, wi_ref,
          bihu_ref, bhhu_ref, bihi_ref, bhhi_ref,
          ublk_ref, iblk_ref, umem_ref, imem_ref,
          nu_ref, ni_ref, uout_ref, iout_ref,
          ue_ref, ie_ref, wa_ref, wb_ref, wc_ref,
          bias_ref, feat_ref, xb_ref, xc_ref, sem_g):
    b = pl.program_id(0)

    @pl.when(b == 0)
    def _compute():
        cps = [pltpu.make_async_copy(umem_ref.at[k, uid_ref[k]], ue_ref.at[k],
                                     sem_g) for k in range(B)]
        cps += [pltpu.make_async_copy(imem_ref.at[k, iid_ref[k]], ie_ref.at[k],
                                      sem_g) for k in range(B)]
        cps += [
            pltpu.make_async_copy(wu_ref.at[:, 0:H], wa_ref.at[0], sem_g),
            pltpu.make_async_copy(wi_ref.at[:, 0:H], wa_ref.at[1], sem_g),
            pltpu.make_async_copy(wu_ref.at[:, H:C0], wb_ref.at[0], sem_g),
            pltpu.make_async_copy(wi_ref.at[:, H:C0], wb_ref.at[1], sem_g),
            pltpu.make_async_copy(wu_ref.at[:, C0:IN], wc_ref.at[0], sem_g),
            pltpu.make_async_copy(wi_ref.at[:, C0:IN], wc_ref.at[1], sem_g),
            pltpu.make_async_copy(bihu_ref, bias_ref.at[0], sem_g),
            pltpu.make_async_copy(bhhu_ref, bias_ref.at[1], sem_g),
            pltpu.make_async_copy(bihi_ref, bias_ref.at[2], sem_g),
            pltpu.make_async_copy(bhhi_ref, bias_ref.at[3], sem_g),
            pltpu.make_async_copy(uf_ref, feat_ref.at[0], sem_g),
            pltpu.make_async_copy(itf_ref, feat_ref.at[1], sem_g),
        ]
        for c in cps:
            c.start()
        for c in cps:
            c.wait()

        ue = ue_ref[...]
        ie = ie_ref[...]
        uf = feat_ref[0]
        itf = feat_ref[1]

        xb_ref[0, :, 0:F] = uf
        xb_ref[0, :, F:H] = ie[:, 0:H - F]
        xc_ref[0, :, 0:F] = ie[:, H - F:H]
        xc_ref[0, :, F:2 * F] = itf
        xb_ref[1, :, 0:F] = itf
        xb_ref[1, :, F:H] = ue[:, 0:H - F]
        xc_ref[1, :, 0:F] = ue[:, H - F:H]
        xc_ref[1, :, F:2 * F] = uf

        def matmul(x, w):
            return lax.dot_general(x, w, (((1,), (1,)), ((), ())),
                                   preferred_element_type=jnp.float32)

        def gru(k, e1):
            gx = (matmul(e1, wa_ref[k]) + matmul(xb_ref[k], wb_ref[k])
                  + matmul(xc_ref[k], wc_ref[k])
                  + bias_ref[2 * k:2 * k + 1, :])
            bhh = bias_ref[2 * k + 1:2 * k + 2, :]
            g = gx + bhh
            r = jax.nn.sigmoid(g[:, :H])
            z = jax.nn.sigmoid(g[:, H:2 * H])
            n = jnp.tanh(gx[:, 2 * H:] + r * bhh[:, 2 * H:])
            out = (1.0 - z) * n
            nrm = jnp.sqrt(jnp.sum(out * out, axis=1, keepdims=True))
            return out / jnp.maximum(nrm, 1e-12)

        nu_ref[...] = gru(0, ue)
        ni_ref[...] = gru(1, ie)

    uout_ref[...] = ublk_ref[...]
    iout_ref[...] = iblk_ref[...]

    uout_ref[0, pl.ds(uid_ref[b], 1), :] = nu_ref[pl.ds(b, 1), :]
    iout_ref[0, pl.ds(iid_ref[b], 1), :] = ni_ref[pl.ds(b, 1), :]


def kernel(user_ids, item_ids, user_features, item_features, user_memory,
           item_memory, W_ih_u, W_hh_u, b_ih_u, b_hh_u, W_ih_i, W_hh_i,
           b_ih_i, b_hh_i):
    del W_hh_u, W_hh_i
    smem = pl.BlockSpec(memory_space=pltpu.SMEM)
    anym = pl.BlockSpec(memory_space=pltpu.MemorySpace.HBM)
    blk = pl.BlockSpec((1, N, H), lambda b: (b, 0, 0))
    f32 = jnp.float32
    return pl.pallas_call(
        _body,
        grid=(B,),
        out_shape=(
            jax.ShapeDtypeStruct((B, H), f32),
            jax.ShapeDtypeStruct((B, H), f32),
            jax.ShapeDtypeStruct((B, N, H), f32),
            jax.ShapeDtypeStruct((B, N, H), f32),
        ),
        in_specs=[smem, smem, anym, anym, anym, anym,
                  anym, anym, anym, anym, blk, blk, anym, anym],
        out_specs=(
            pl.BlockSpec((B, H), lambda b: (0, 0)),
            pl.BlockSpec((B, H), lambda b: (0, 0)),
            blk,
            blk,
        ),
        scratch_shapes=[
            pltpu.VMEM((B, H), f32),
            pltpu.VMEM((B, H), f32),
            pltpu.VMEM((2, G3, H), f32),
            pltpu.VMEM((2, G3, H), f32),
            pltpu.VMEM((2, G3, 2 * F), f32),
            pltpu.VMEM((4, G3), f32),
            pltpu.VMEM((2, B, F), f32),
            pltpu.VMEM((2, B, H), f32),
            pltpu.VMEM((2, B, 2 * F), f32),
            pltpu.SemaphoreType.DMA,
        ],
    )(user_ids, item_ids, user_features, item_features, W_ih_u, W_ih_i,
      b_ih_u, b_hh_u, b_ih_i, b_hh_i,
      user_memory, item_memory, user_memory, item_memory)
